# Initial kernel scaffold; baseline (speedup 1.0000x reference)
import jax
import jax.numpy as jnp
from jax.experimental import pallas as pl


def kernel(x, edge_index, edge_attr, batch, demand, capacity, n_steps, greedy, T, W_node, W_edge, W_lay, a_src, a_dst, a_edge, W_k, W_ctx):
    raise NotImplementedError("write your pallas kernel here")


# speedup vs baseline: 1.0144x; 1.0144x over previous
import functools

import jax
import jax.numpy as jnp
import numpy as np
from jax.experimental import pallas as pl
from jax.experimental.pallas import tpu as pltpu

N_NODES = 10000
HID = 128
NEG = 0.2


def _leaky(v, slope):
    return jnp.where(v >= 0, v, slope * v)


def _mm_body(a_ref, b_ref, o_ref):
    o_ref[...] = jnp.dot(a_ref[...], b_ref[...],
                         preferred_element_type=jnp.float32)


def _mm(a, b, bm=400):
    M, K = a.shape
    K2, N = b.shape
    assert K == K2 and M % bm == 0
    return pl.pallas_call(
        _mm_body,
        grid=(M // bm,),
        in_specs=[pl.BlockSpec((bm, K), lambda i: (i, 0)),
                  pl.BlockSpec((K, N), lambda i: (0, 0))],
        out_specs=pl.BlockSpec((bm, N), lambda i: (i, 0)),
        out_shape=jax.ShapeDtypeStruct((M, N), jnp.float32),
    )(a, b)


def _encode(x, edge_index, edge_attr, W_node, W_edge, W_lay, a_src, a_dst, a_edge):
    N = x.shape[0]
    h = _mm(x, W_node)
    e = edge_attr @ W_edge
    src = edge_index[0]
    dst = edge_index[1]
    for l in range(W_lay.shape[0]):
        hs = _mm(h, W_lay[l])
        sc = _leaky(hs[src] @ a_src[l] + hs[dst] @ a_dst[l] + e @ a_edge[l], 0.2)
        m = jax.ops.segment_max(sc, dst, num_segments=N)
        m = jnp.where(jnp.isfinite(m), m, 0.0)
        ex = jnp.exp(sc - m[dst])
        den = jax.ops.segment_sum(ex, dst, num_segments=N)
        alpha = ex / (den[dst] + 1e-16)
        agg = jax.ops.segment_sum(alpha[:, None] * hs[src], dst, num_segments=N)
        h = h + _leaky(agg, NEG)
    return h


def _decode(xb, g, cap0, dem, n_steps, n_alloc, T, greedy, W_k, W_ctx):
    Bsz, Nn, Hd = xb.shape
    keys_ = xb @ W_k
    visited = jnp.zeros((Bsz, Nn), dtype=bool)
    cur = xb[:, 0, :]
    cap = cap0
    scale = 1.0 / np.sqrt(float(Hd))
    acts = jnp.zeros((Bsz, n_alloc), dtype=jnp.int32)
    lps = jnp.zeros((Bsz, n_alloc), dtype=xb.dtype)

    def body(t, carry):
        visited, cur, cap, acts, lps = carry
        ctx = jnp.concatenate([g, cur, cap], axis=-1)
        q = ctx @ W_ctx
        sc = jnp.einsum('bh,bnh->bn', q, keys_) * scale
        sc = 10.0 * jnp.tanh(sc) / T
        sc = jnp.where(visited, -1e9, sc)
        lp = jax.nn.log_softmax(sc, axis=-1)
        a = jnp.argmax(lp, axis=-1)
        acts = acts.at[:, t].set(a)
        lps = lps.at[:, t].set(jnp.take_along_axis(lp, a[:, None], axis=1)[:, 0])
        visited = visited.at[jnp.arange(Bsz), a].set(True)
        dsel = jnp.take_along_axis(dem, a[:, None], axis=1)
        cap = jnp.maximum(cap - dsel, 0.0)
        cur = xb[jnp.arange(Bsz), a]
        return (visited, cur, cap, acts, lps)

    visited, cur, cap, acts, lps = jax.lax.fori_loop(
        0, n_steps, body, (visited, cur, cap, acts, lps))
    return acts, lps


def kernel(x, edge_index, edge_attr, batch, demand, capacity, n_steps, greedy,
           T, W_node, W_edge, W_lay, a_src, a_dst, a_edge, W_k, W_ctx):
    h = _encode(x, edge_index, edge_attr, W_node, W_edge, W_lay,
                a_src, a_dst, a_edge)
    Bsz = capacity.shape[0]
    xb = h.reshape(Bsz, -1, h.shape[-1])
    g = xb.mean(axis=1)
    dem = jnp.where(batch >= 0, demand, 0.0).reshape(Bsz, -1)
    cap0 = capacity.reshape(Bsz, -1)[:, :1]
    actions, log_p = _decode(xb, g, cap0, dem, 100, 100, T, greedy, W_k, W_ctx)
    return (actions, log_p)


# speedup vs baseline: 1.1806x; 1.1639x over previous
import functools

import jax
import jax.numpy as jnp
import numpy as np
from jax.experimental import pallas as pl
from jax.experimental.pallas import tpu as pltpu

N_NODES = 10000
HID = 128
NEG = 0.2


def _leaky(v, slope):
    return jnp.where(v >= 0, v, slope * v)


def _mm_body(a_ref, b_ref, o_ref):
    o_ref[...] = jnp.dot(a_ref[...], b_ref[...],
                         preferred_element_type=jnp.float32)


def _mm(a, b, bm=400):
    M, K = a.shape
    K2, N = b.shape
    assert K == K2 and M % bm == 0
    return pl.pallas_call(
        _mm_body,
        grid=(M // bm,),
        in_specs=[pl.BlockSpec((bm, K), lambda i: (i, 0)),
                  pl.BlockSpec((K, N), lambda i: (0, 0))],
        out_specs=pl.BlockSpec((bm, N), lambda i: (i, 0)),
        out_shape=jax.ShapeDtypeStruct((M, N), jnp.float32),
    )(a, b)


def _encode(x, edge_index, edge_attr, W_node, W_edge, W_lay, a_src, a_dst, a_edge):
    N = x.shape[0]
    h = _mm(x, W_node)
    e = edge_attr @ W_edge
    src = edge_index[0]
    dst = edge_index[1]
    for l in range(W_lay.shape[0]):
        hs = _mm(h, W_lay[l])
        sc = _leaky(hs[src] @ a_src[l] + hs[dst] @ a_dst[l] + e @ a_edge[l], 0.2)
        m = jax.ops.segment_max(sc, dst, num_segments=N)
        m = jnp.where(jnp.isfinite(m), m, 0.0)
        ex = jnp.exp(sc - m[dst])
        den = jax.ops.segment_sum(ex, dst, num_segments=N)
        alpha = ex / (den[dst] + 1e-16)
        agg = jax.ops.segment_sum(alpha[:, None] * hs[src], dst, num_segments=N)
        h = h + _leaky(agg, NEG)
    return h


B = 100
NN = 100
SCALE = 1.0 / np.sqrt(128.0)


def _dec_body(h3_ref, g_ref, keys_ref, dem_ref, cap0_ref, wctx_ref, T_ref,
              acts_ref, lps_ref):
    h3 = h3_ref[...]
    g = g_ref[...]
    keys = keys_ref[...]
    dem = dem_ref[...]
    wctx = wctx_ref[...]
    Tval = T_ref[0, 0]
    col = jax.lax.broadcasted_iota(jnp.int32, (B, NN), 1)

    def body(t, carry):
        visited, cur, cap, acts, lps = carry
        ctx = jnp.concatenate([g, cur, cap], axis=-1)
        q = jnp.dot(ctx, wctx, preferred_element_type=jnp.float32)
        sc = jax.lax.dot_general(q, keys, (((1,), (2,)), ((0,), (0,))),
                                 preferred_element_type=jnp.float32) * SCALE
        sc = 10.0 * jnp.tanh(sc) / Tval
        sc = jnp.where(visited > 0, -1e9, sc)
        mx = jnp.max(sc, axis=-1, keepdims=True)
        sh = sc - mx
        lp = sh - jnp.log(jnp.sum(jnp.exp(sh), axis=-1, keepdims=True))
        lmax = jnp.max(lp, axis=-1, keepdims=True)
        a = jnp.min(jnp.where(lp == lmax, col, NN), axis=-1, keepdims=True)
        onehot = (col == a).astype(jnp.float32)
        acts = jnp.where(col == t, a, acts)
        lps_sel = jnp.sum(lp * onehot, axis=-1, keepdims=True)
        lps = jnp.where(col == t, lps_sel, lps)
        visited = jnp.maximum(visited, onehot)
        dsel = jnp.sum(dem * onehot, axis=-1, keepdims=True)
        cap = jnp.maximum(cap - dsel, 0.0)
        cur = jnp.sum(h3 * onehot[:, :, None], axis=1)
        return (visited, cur, cap, acts, lps)

    visited0 = jnp.zeros((B, NN), jnp.float32)
    cur0 = h3[:, 0, :]
    cap0 = cap0_ref[...]
    acts0 = jnp.zeros((B, NN), jnp.int32)
    lps0 = jnp.zeros((B, NN), jnp.float32)
    _, _, _, acts, lps = jax.lax.fori_loop(
        0, NN, body, (visited0, cur0, cap0, acts0, lps0))
    acts_ref[...] = acts
    lps_ref[...] = lps


def _decode_pallas(h, g, dem, cap0, T, W_k, W_ctx):
    h3 = h.reshape(B, NN, HID)
    keys = _mm(h, W_k).reshape(B, NN, HID)
    Tarr = jnp.full((1, 1), T, jnp.float32)
    return pl.pallas_call(
        _dec_body,
        out_shape=(jax.ShapeDtypeStruct((B, NN), jnp.int32),
                   jax.ShapeDtypeStruct((B, NN), jnp.float32)),
    )(h3, g, keys, dem, cap0, W_ctx, Tarr)


def kernel(x, edge_index, edge_attr, batch, demand, capacity, n_steps, greedy,
           T, W_node, W_edge, W_lay, a_src, a_dst, a_edge, W_k, W_ctx):
    h = _encode(x, edge_index, edge_attr, W_node, W_edge, W_lay,
                a_src, a_dst, a_edge)
    Bsz = capacity.shape[0]
    g = h.reshape(Bsz, -1, h.shape[-1]).mean(axis=1)
    dem = jnp.where(batch >= 0, demand, 0.0).reshape(Bsz, -1)
    cap0 = capacity.reshape(Bsz, -1)[:, :1]
    actions, log_p = _decode_pallas(h, g, dem, cap0, T, W_k, W_ctx)
    return (actions, log_p)
